# Initial kernel scaffold; baseline (speedup 1.0000x reference)
import jax
import jax.numpy as jnp
from jax.experimental import pallas as pl


def kernel(x, edge_index, edge_weight, W1r, b1r, W1n, g1, beta1, W2r, b2r, W2n, g2, beta2, W3r, b3r, W3n, g3, beta3, Wc1, bc1, Wc2, bc2):
    raise NotImplementedError("write your pallas kernel here")


# speedup vs baseline: 3.4642x; 3.4642x over previous
import functools

import jax
import jax.numpy as jnp
from jax import lax
from jax.experimental import pallas as pl
from jax.experimental.pallas import tpu as pltpu
from jax.experimental.pallas import tpu_sc as plsc

N = 50000
E = 800000
BR = 400
NB = N // BR
CHUNK = 128
N_TILES = 16
N_CHUNKS = 392
EDGES_PER_TILE = N_CHUNKS * CHUNK
E_PAD = N_TILES * EDGES_PER_TILE
ROWS_PER_TILE = 3200
N_ACC = N_TILES * ROWS_PER_TILE
F32 = jnp.float32


def _sc_agg_body(use_ew, table, src, dst, ew, out, src_i, dst_i, ew_b, msg,
                 acc, sem):
    c = lax.axis_index("c")
    s = lax.axis_index("s")
    r0 = s * ROWS_PER_TILE

    def _zrow(i, _):
        msg[i, pl.ds(0, 16)] = jnp.zeros((16,), F32)
        msg[i, pl.ds(16, 16)] = jnp.zeros((16,), F32)
        return 0
    lax.fori_loop(0, CHUNK, _zrow, 0)

    def _zacc(k, _):
        pltpu.sync_copy(msg, acc.at[pl.ds(r0 + k * CHUNK, CHUNK)])
        return 0
    lax.fori_loop(0, ROWS_PER_TILE // CHUNK, _zacc, 0)
    plsc.subcore_barrier()

    base = s * EDGES_PER_TILE
    coff = c * N

    def _chunk(j, _):
        off = base + j * CHUNK
        pltpu.sync_copy(src.at[pl.ds(off, CHUNK)], src_i)
        pltpu.sync_copy(dst.at[pl.ds(off, CHUNK)], dst_i)
        for f in range(CHUNK // 16):
            sl = pl.ds(f * 16, 16)
            src_i[sl] = src_i[sl] + coff
        pltpu.async_copy(table.at[src_i], msg, sem).wait()
        if use_ew:
            pltpu.sync_copy(ew.at[pl.ds(off, CHUNK)], ew_b)
            for g in range(CHUNK // 16):
                w16 = ew_b[pl.ds(g * 16, 16)]
                for el in range(16):
                    e = g * 16 + el
                    wv = jnp.full((16,), w16[el], F32)
                    msg[e, pl.ds(0, 16)] = msg[e, pl.ds(0, 16)] * wv
                    msg[e, pl.ds(16, 16)] = msg[e, pl.ds(16, 16)] * wv
        pltpu.sync_copy(msg, acc.at[dst_i], add=True)
        return 0
    lax.fori_loop(0, N_CHUNKS, _chunk, 0)
    plsc.subcore_barrier()

    def _drain(k, _):
        r = r0 + k * CHUNK
        pltpu.sync_copy(acc.at[pl.ds(r, CHUNK)], msg)
        pltpu.sync_copy(msg, out.at[c, pl.ds(r, CHUNK)])
        return 0
    lax.fori_loop(0, ROWS_PER_TILE // CHUNK, _drain, 0)


@functools.cache
def _build_sc_agg(use_ew):
    mesh = plsc.VectorSubcoreMesh(core_axis_name="c", subcore_axis_name="s",
                                  num_cores=2, num_subcores=N_TILES)
    return pl.kernel(
        functools.partial(_sc_agg_body, use_ew),
        out_type=jax.ShapeDtypeStruct((2, N_ACC, 32), F32),
        mesh=mesh,
        scratch_types=[
            pltpu.VMEM((CHUNK,), jnp.int32),
            pltpu.VMEM((CHUNK,), jnp.int32),
            pltpu.VMEM((CHUNK,), F32),
            pltpu.VMEM((CHUNK, 32), F32),
            pltpu.VMEM_SHARED((N_ACC, 32), F32),
            pltpu.SemaphoreType.DMA,
        ],
        compiler_params=pltpu.CompilerParams(use_tc_tiling_on_sc=False),
    )


def _tc1_body(x_ref, wr_ref, wn_ref, y_ref, xn_ref):
    xb = x_ref[...]
    y = jnp.dot(xb, wr_ref[...], preferred_element_type=F32)
    y_ref[0] = y[:, :32]
    y_ref[1] = y[:, 32:]
    xn_ref[...] = jnp.dot(xb, wn_ref[...], preferred_element_type=F32)


def _build_tc1(interpret=False):
    return pl.pallas_call(
        _tc1_body,
        grid=(NB,),
        in_specs=[
            pl.BlockSpec((BR, 128), lambda i: (i, 0)),
            pl.BlockSpec((128, 64), lambda i: (0, 0)),
            pl.BlockSpec((128, 64), lambda i: (0, 0)),
        ],
        out_specs=[
            pl.BlockSpec((2, BR, 32), lambda i: (0, i, 0)),
            pl.BlockSpec((BR, 64), lambda i: (i, 0)),
        ],
        out_shape=[
            jax.ShapeDtypeStruct((2, N, 32), F32),
            jax.ShapeDtypeStruct((N, 64), F32),
        ],
        interpret=interpret,
    )


def _sum_stats_body(z, st_ref, i):
    @pl.when(i == 0)
    def _():
        st_ref[...] = jnp.zeros_like(st_ref)
    st_ref[...] += jnp.stack([jnp.sum(z, 0), jnp.sum(z * z, 0)])


def _tc2_body(agg_ref, xn_ref, b_ref, z_ref, st_ref):
    z = jnp.concatenate([agg_ref[0], agg_ref[1]], axis=1) + xn_ref[...] \
        + b_ref[...]
    z_ref[...] = z
    _sum_stats_body(z, st_ref, pl.program_id(0))


def _build_tc2(d, interpret=False):
    return pl.pallas_call(
        _tc2_body,
        grid=(NB,),
        in_specs=[
            pl.BlockSpec((2, BR, 32), lambda i: (0, i, 0)),
            pl.BlockSpec((BR, d), lambda i: (i, 0)),
            pl.BlockSpec((1, d), lambda i: (0, 0)),
        ],
        out_specs=[
            pl.BlockSpec((BR, d), lambda i: (i, 0)),
            pl.BlockSpec((2, d), lambda i: (0, 0)),
        ],
        out_shape=[
            jax.ShapeDtypeStruct((N, d), F32),
            jax.ShapeDtypeStruct((2, d), F32),
        ],
        interpret=interpret,
    )


def _bn_relu(z, st, g, b):
    m = st[0] / N
    v = st[1] / N - m * m
    r = lax.rsqrt(v + 1e-5)
    return jnp.maximum((z - m) * r * g + b, 0.0)


def _tc3_body(z_ref, st_ref, g_ref, b_ref, w_ref, h_ref, zp_ref):
    h = _bn_relu(z_ref[...], st_ref[...], g_ref[0], b_ref[0])
    h_ref[0] = h[:, :32]
    h_ref[1] = h[:, 32:]
    zp_ref[...] = jnp.dot(h, w_ref[...], preferred_element_type=F32)


def _build_tc3(interpret=False):
    return pl.pallas_call(
        _tc3_body,
        grid=(NB,),
        in_specs=[
            pl.BlockSpec((BR, 64), lambda i: (i, 0)),
            pl.BlockSpec((2, 64), lambda i: (0, 0)),
            pl.BlockSpec((1, 64), lambda i: (0, 0)),
            pl.BlockSpec((1, 64), lambda i: (0, 0)),
            pl.BlockSpec((64, 128), lambda i: (0, 0)),
        ],
        out_specs=[
            pl.BlockSpec((2, BR, 32), lambda i: (0, i, 0)),
            pl.BlockSpec((BR, 128), lambda i: (i, 0)),
        ],
        out_shape=[
            jax.ShapeDtypeStruct((2, N, 32), F32),
            jax.ShapeDtypeStruct((N, 128), F32),
        ],
        interpret=interpret,
    )


def _tc4_body(agg_ref, zp_ref, wr_ref, b_ref, z_ref, st_ref):
    a = jnp.concatenate([agg_ref[0], agg_ref[1]], axis=1)
    z = jnp.dot(a, wr_ref[...], preferred_element_type=F32) + zp_ref[...] \
        + b_ref[...]
    z_ref[...] = z
    _sum_stats_body(z, st_ref, pl.program_id(0))


def _build_tc4(interpret=False):
    return pl.pallas_call(
        _tc4_body,
        grid=(NB,),
        in_specs=[
            pl.BlockSpec((2, BR, 32), lambda i: (0, i, 0)),
            pl.BlockSpec((BR, 128), lambda i: (i, 0)),
            pl.BlockSpec((64, 128), lambda i: (0, 0)),
            pl.BlockSpec((1, 128), lambda i: (0, 0)),
        ],
        out_specs=[
            pl.BlockSpec((BR, 128), lambda i: (i, 0)),
            pl.BlockSpec((2, 128), lambda i: (0, 0)),
        ],
        out_shape=[
            jax.ShapeDtypeStruct((N, 128), F32),
            jax.ShapeDtypeStruct((2, 128), F32),
        ],
        interpret=interpret,
    )


def _tc5_body(z_ref, st_ref, g_ref, b_ref, wr_ref, wn_ref, y_ref, xn_ref):
    h = _bn_relu(z_ref[...], st_ref[...], g_ref[0], b_ref[0])
    y = jnp.dot(h, wr_ref[...], preferred_element_type=F32)
    y_ref[0] = y[:, :32]
    y_ref[1] = y[:, 32:]
    xn_ref[...] = jnp.dot(h, wn_ref[...], preferred_element_type=F32)


def _build_tc5(interpret=False):
    return pl.pallas_call(
        _tc5_body,
        grid=(NB,),
        in_specs=[
            pl.BlockSpec((BR, 128), lambda i: (i, 0)),
            pl.BlockSpec((2, 128), lambda i: (0, 0)),
            pl.BlockSpec((1, 128), lambda i: (0, 0)),
            pl.BlockSpec((1, 128), lambda i: (0, 0)),
            pl.BlockSpec((128, 64), lambda i: (0, 0)),
            pl.BlockSpec((128, 64), lambda i: (0, 0)),
        ],
        out_specs=[
            pl.BlockSpec((2, BR, 32), lambda i: (0, i, 0)),
            pl.BlockSpec((BR, 64), lambda i: (i, 0)),
        ],
        out_shape=[
            jax.ShapeDtypeStruct((2, N, 32), F32),
            jax.ShapeDtypeStruct((N, 64), F32),
        ],
        interpret=interpret,
    )


def _tc7_body(z_ref, st_ref, g_ref, b_ref, h1_ref, wc1_ref, bc1_ref, wc2_ref,
              bc2_ref, out_ref):
    st = st_ref[...]
    m = st[0] / N
    v = st[1] / N - m * m
    r = lax.rsqrt(v + 1e-5)
    bn = (z_ref[...] - m) * r * g_ref[0] + b_ref[0]
    h1 = jnp.concatenate([h1_ref[0], h1_ref[1]], axis=1)
    h3 = jnp.maximum(bn + h1, 0.0)
    t = jnp.maximum(jnp.dot(h3, wc1_ref[...], preferred_element_type=F32)
                    + bc1_ref[0], 0.0)
    out_ref[...] = jnp.dot(t, wc2_ref[...], preferred_element_type=F32) \
        + bc2_ref[0]


def _build_tc7(interpret=False):
    return pl.pallas_call(
        _tc7_body,
        grid=(NB,),
        in_specs=[
            pl.BlockSpec((BR, 64), lambda i: (i, 0)),
            pl.BlockSpec((2, 64), lambda i: (0, 0)),
            pl.BlockSpec((1, 64), lambda i: (0, 0)),
            pl.BlockSpec((1, 64), lambda i: (0, 0)),
            pl.BlockSpec((2, BR, 32), lambda i: (0, i, 0)),
            pl.BlockSpec((64, 32), lambda i: (0, 0)),
            pl.BlockSpec((1, 32), lambda i: (0, 0)),
            pl.BlockSpec((32, 2), lambda i: (0, 0)),
            pl.BlockSpec((1, 2), lambda i: (0, 0)),
        ],
        out_specs=pl.BlockSpec((BR, 2), lambda i: (i, 0)),
        out_shape=jax.ShapeDtypeStruct((N, 2), F32),
        interpret=interpret,
    )


_tc1 = _build_tc1()
_tc2 = _build_tc2(64)
_tc3 = _build_tc3()
_tc4 = _build_tc4()
_tc5 = _build_tc5()
_tc6 = _build_tc2(64)
_tc7 = _build_tc7()


def kernel(x, edge_index, edge_weight, W1r, b1r, W1n, g1, beta1, W2r, b2r,
           W2n, g2, beta2, W3r, b3r, W3n, g3, beta3, Wc1, bc1, Wc2, bc2):
    pad = E_PAD - E
    src = jnp.concatenate([edge_index[0], jnp.zeros((pad,), jnp.int32)])
    dst = jnp.concatenate(
        [edge_index[1], N + (jnp.arange(pad, dtype=jnp.int32) % 16)])
    ew = jnp.concatenate([edge_weight, jnp.zeros((pad,), F32)])

    sc_plain = _build_sc_agg(False)
    sc_weighted = _build_sc_agg(True)
    y1, xn1 = _tc1(x, W1r, W1n)
    agg1 = sc_plain(y1.reshape(2 * N, 32), src, dst, ew)
    z1, st1 = _tc2(agg1, xn1, b1r[None])
    h1, z2p = _tc3(z1, st1, g1[None], beta1[None], W2n)
    agg2 = sc_plain(h1.reshape(2 * N, 32), src, dst, ew)
    z2, st2 = _tc4(agg2, z2p, W2r, b2r[None])
    y3, xn3 = _tc5(z2, st2, g2[None], beta2[None], W3r, W3n)
    agg3 = sc_weighted(y3.reshape(2 * N, 32), src, dst, ew)
    z3, st3 = _tc6(agg3, xn3, b3r[None])
    return _tc7(z3, st3, g3[None], beta3[None], h1, Wc1, bc1[None], Wc2,
                bc2[None])


# speedup vs baseline: 6.0414x; 1.7440x over previous
import functools

import jax
import jax.numpy as jnp
from jax import lax
from jax.experimental import pallas as pl
from jax.experimental.pallas import tpu as pltpu
from jax.experimental.pallas import tpu_sc as plsc

N = 50000
E = 800000
BR = 400
NB = N // BR
CHUNK = 128
N_TILES = 16
N_CHUNKS = 392
EDGES_PER_TILE = N_CHUNKS * CHUNK
E_PAD = N_TILES * EDGES_PER_TILE
ROWS_PER_TILE = 3200
N_ACC = N_TILES * ROWS_PER_TILE
F32 = jnp.float32


def _sc_agg_body(use_ew, table, src, dst, ew, out, src_i, dst_i, ew_b, msg,
                 acc, ssem, dsem, esem, gsem):
    c = lax.axis_index("c")
    s = lax.axis_index("s")
    r0 = s * ROWS_PER_TILE

    def _zrow(i, _):
        msg[0, i, pl.ds(0, 16)] = jnp.zeros((16,), F32)
        msg[0, i, pl.ds(16, 16)] = jnp.zeros((16,), F32)
        return 0
    lax.fori_loop(0, CHUNK, _zrow, 0)

    def _zacc(k, _):
        pltpu.sync_copy(msg.at[0], acc.at[pl.ds(r0 + k * CHUNK, CHUNK)])
        return 0
    lax.fori_loop(0, ROWS_PER_TILE // CHUNK, _zacc, 0)
    plsc.subcore_barrier()

    base = s * EDGES_PER_TILE
    coff = c * N

    def start_idx(j, b):
        off = base + j * CHUNK
        pltpu.async_copy(src.at[pl.ds(off, CHUNK)], src_i.at[b], ssem.at[b])
        pltpu.async_copy(dst.at[pl.ds(off, CHUNK)], dst_i.at[b], dsem.at[b])
        if use_ew:
            pltpu.async_copy(ew.at[pl.ds(off, CHUNK)], ew_b.at[b],
                             esem.at[b])

    def gather_chunk(b):
        pltpu.make_async_copy(src.at[pl.ds(base, CHUNK)], src_i.at[b],
                              ssem.at[b]).wait()
        for f in range(CHUNK // 16):
            sl = pl.ds(f * 16, 16)
            src_i[b, sl] = src_i[b, sl] + coff
        pltpu.async_copy(table.at[src_i.at[b]], msg.at[b], gsem.at[b])

    def wait_gather(b):
        pltpu.make_async_copy(table.at[src_i.at[b]], msg.at[b],
                              gsem.at[b]).wait()

    def do_scatter(b):
        if use_ew:
            pltpu.make_async_copy(ew.at[pl.ds(base, CHUNK)], ew_b.at[b],
                                  esem.at[b]).wait()
            for g in range(CHUNK // 16):
                w16 = ew_b[b, pl.ds(g * 16, 16)]
                for el in range(16):
                    e = g * 16 + el
                    wv = jnp.full((16,), w16[el], F32)
                    msg[b, e, pl.ds(0, 16)] = msg[b, e, pl.ds(0, 16)] * wv
                    msg[b, e, pl.ds(16, 16)] = msg[b, e, pl.ds(16, 16)] * wv
        pltpu.make_async_copy(dst.at[pl.ds(base, CHUNK)], dst_i.at[b],
                              dsem.at[b]).wait()
        pltpu.sync_copy(msg.at[b], acc.at[dst_i.at[b]], add=True)

    start_idx(0, 0)
    start_idx(1, 1)
    gather_chunk(0)

    def _outer(t, _):
        j0 = 2 * t
        for b in range(2):
            wait_gather(b)
            gather_chunk(1 - b)
            do_scatter(b)
            start_idx(j0 + b + 2, b)
        return 0
    lax.fori_loop(0, (N_CHUNKS - 2) // 2, _outer, 0)
    wait_gather(0)
    gather_chunk(1)
    do_scatter(0)
    wait_gather(1)
    do_scatter(1)
    plsc.subcore_barrier()

    def _drain(k, _):
        r = r0 + k * CHUNK
        pltpu.sync_copy(acc.at[pl.ds(r, CHUNK)], msg.at[0])
        pltpu.sync_copy(msg.at[0], out.at[c, pl.ds(r, CHUNK)])
        return 0
    lax.fori_loop(0, ROWS_PER_TILE // CHUNK, _drain, 0)


@functools.cache
def _build_sc_agg(use_ew):
    mesh = plsc.VectorSubcoreMesh(core_axis_name="c", subcore_axis_name="s",
                                  num_cores=2, num_subcores=N_TILES)
    return pl.kernel(
        functools.partial(_sc_agg_body, use_ew),
        out_type=jax.ShapeDtypeStruct((2, N_ACC, 32), F32),
        mesh=mesh,
        scratch_types=[
            pltpu.VMEM((2, CHUNK), jnp.int32),
            pltpu.VMEM((2, CHUNK), jnp.int32),
            pltpu.VMEM((2, CHUNK), F32),
            pltpu.VMEM((2, CHUNK, 32), F32),
            pltpu.VMEM_SHARED((N_ACC, 32), F32),
            pltpu.SemaphoreType.DMA((2,)),
            pltpu.SemaphoreType.DMA((2,)),
            pltpu.SemaphoreType.DMA((2,)),
            pltpu.SemaphoreType.DMA((2,)),
        ],
        compiler_params=pltpu.CompilerParams(use_tc_tiling_on_sc=False),
    )


def _tc1_body(x_ref, wr_ref, wn_ref, y_ref, xn_ref):
    xb = x_ref[...]
    y = jnp.dot(xb, wr_ref[...], preferred_element_type=F32)
    y_ref[0] = y[:, :32]
    y_ref[1] = y[:, 32:]
    xn_ref[...] = jnp.dot(xb, wn_ref[...], preferred_element_type=F32)


def _build_tc1(interpret=False):
    return pl.pallas_call(
        _tc1_body,
        grid=(NB,),
        in_specs=[
            pl.BlockSpec((BR, 128), lambda i: (i, 0)),
            pl.BlockSpec((128, 64), lambda i: (0, 0)),
            pl.BlockSpec((128, 64), lambda i: (0, 0)),
        ],
        out_specs=[
            pl.BlockSpec((2, BR, 32), lambda i: (0, i, 0)),
            pl.BlockSpec((BR, 64), lambda i: (i, 0)),
        ],
        out_shape=[
            jax.ShapeDtypeStruct((2, N, 32), F32),
            jax.ShapeDtypeStruct((N, 64), F32),
        ],
        interpret=interpret,
    )


def _sum_stats_body(z, st_ref, i):
    @pl.when(i == 0)
    def _():
        st_ref[...] = jnp.zeros_like(st_ref)
    st_ref[...] += jnp.stack([jnp.sum(z, 0), jnp.sum(z * z, 0)])


def _tc2_body(agg_ref, xn_ref, b_ref, z_ref, st_ref):
    z = jnp.concatenate([agg_ref[0], agg_ref[1]], axis=1) + xn_ref[...] \
        + b_ref[...]
    z_ref[...] = z
    _sum_stats_body(z, st_ref, pl.program_id(0))


def _build_tc2(d, interpret=False):
    return pl.pallas_call(
        _tc2_body,
        grid=(NB,),
        in_specs=[
            pl.BlockSpec((2, BR, 32), lambda i: (0, i, 0)),
            pl.BlockSpec((BR, d), lambda i: (i, 0)),
            pl.BlockSpec((1, d), lambda i: (0, 0)),
        ],
        out_specs=[
            pl.BlockSpec((BR, d), lambda i: (i, 0)),
            pl.BlockSpec((2, d), lambda i: (0, 0)),
        ],
        out_shape=[
            jax.ShapeDtypeStruct((N, d), F32),
            jax.ShapeDtypeStruct((2, d), F32),
        ],
        interpret=interpret,
    )


def _bn_relu(z, st, g, b):
    m = st[0] / N
    v = st[1] / N - m * m
    r = lax.rsqrt(v + 1e-5)
    return jnp.maximum((z - m) * r * g + b, 0.0)


def _tc3_body(z_ref, st_ref, g_ref, b_ref, w_ref, h_ref, zp_ref):
    h = _bn_relu(z_ref[...], st_ref[...], g_ref[0], b_ref[0])
    h_ref[0] = h[:, :32]
    h_ref[1] = h[:, 32:]
    zp_ref[...] = jnp.dot(h, w_ref[...], preferred_element_type=F32)


def _build_tc3(interpret=False):
    return pl.pallas_call(
        _tc3_body,
        grid=(NB,),
        in_specs=[
            pl.BlockSpec((BR, 64), lambda i: (i, 0)),
            pl.BlockSpec((2, 64), lambda i: (0, 0)),
            pl.BlockSpec((1, 64), lambda i: (0, 0)),
            pl.BlockSpec((1, 64), lambda i: (0, 0)),
            pl.BlockSpec((64, 128), lambda i: (0, 0)),
        ],
        out_specs=[
            pl.BlockSpec((2, BR, 32), lambda i: (0, i, 0)),
            pl.BlockSpec((BR, 128), lambda i: (i, 0)),
        ],
        out_shape=[
            jax.ShapeDtypeStruct((2, N, 32), F32),
            jax.ShapeDtypeStruct((N, 128), F32),
        ],
        interpret=interpret,
    )


def _tc4_body(agg_ref, zp_ref, wr_ref, b_ref, z_ref, st_ref):
    a = jnp.concatenate([agg_ref[0], agg_ref[1]], axis=1)
    z = jnp.dot(a, wr_ref[...], preferred_element_type=F32) + zp_ref[...] \
        + b_ref[...]
    z_ref[...] = z
    _sum_stats_body(z, st_ref, pl.program_id(0))


def _build_tc4(interpret=False):
    return pl.pallas_call(
        _tc4_body,
        grid=(NB,),
        in_specs=[
            pl.BlockSpec((2, BR, 32), lambda i: (0, i, 0)),
            pl.BlockSpec((BR, 128), lambda i: (i, 0)),
            pl.BlockSpec((64, 128), lambda i: (0, 0)),
            pl.BlockSpec((1, 128), lambda i: (0, 0)),
        ],
        out_specs=[
            pl.BlockSpec((BR, 128), lambda i: (i, 0)),
            pl.BlockSpec((2, 128), lambda i: (0, 0)),
        ],
        out_shape=[
            jax.ShapeDtypeStruct((N, 128), F32),
            jax.ShapeDtypeStruct((2, 128), F32),
        ],
        interpret=interpret,
    )


def _tc5_body(z_ref, st_ref, g_ref, b_ref, wr_ref, wn_ref, y_ref, xn_ref):
    h = _bn_relu(z_ref[...], st_ref[...], g_ref[0], b_ref[0])
    y = jnp.dot(h, wr_ref[...], preferred_element_type=F32)
    y_ref[0] = y[:, :32]
    y_ref[1] = y[:, 32:]
    xn_ref[...] = jnp.dot(h, wn_ref[...], preferred_element_type=F32)


def _build_tc5(interpret=False):
    return pl.pallas_call(
        _tc5_body,
        grid=(NB,),
        in_specs=[
            pl.BlockSpec((BR, 128), lambda i: (i, 0)),
            pl.BlockSpec((2, 128), lambda i: (0, 0)),
            pl.BlockSpec((1, 128), lambda i: (0, 0)),
            pl.BlockSpec((1, 128), lambda i: (0, 0)),
            pl.BlockSpec((128, 64), lambda i: (0, 0)),
            pl.BlockSpec((128, 64), lambda i: (0, 0)),
        ],
        out_specs=[
            pl.BlockSpec((2, BR, 32), lambda i: (0, i, 0)),
            pl.BlockSpec((BR, 64), lambda i: (i, 0)),
        ],
        out_shape=[
            jax.ShapeDtypeStruct((2, N, 32), F32),
            jax.ShapeDtypeStruct((N, 64), F32),
        ],
        interpret=interpret,
    )


def _tc7_body(z_ref, st_ref, g_ref, b_ref, h1_ref, wc1_ref, bc1_ref, wc2_ref,
              bc2_ref, out_ref):
    st = st_ref[...]
    m = st[0] / N
    v = st[1] / N - m * m
    r = lax.rsqrt(v + 1e-5)
    bn = (z_ref[...] - m) * r * g_ref[0] + b_ref[0]
    h1 = jnp.concatenate([h1_ref[0], h1_ref[1]], axis=1)
    h3 = jnp.maximum(bn + h1, 0.0)
    t = jnp.maximum(jnp.dot(h3, wc1_ref[...], preferred_element_type=F32)
                    + bc1_ref[0], 0.0)
    out_ref[...] = jnp.dot(t, wc2_ref[...], preferred_element_type=F32) \
        + bc2_ref[0]


def _build_tc7(interpret=False):
    return pl.pallas_call(
        _tc7_body,
        grid=(NB,),
        in_specs=[
            pl.BlockSpec((BR, 64), lambda i: (i, 0)),
            pl.BlockSpec((2, 64), lambda i: (0, 0)),
            pl.BlockSpec((1, 64), lambda i: (0, 0)),
            pl.BlockSpec((1, 64), lambda i: (0, 0)),
            pl.BlockSpec((2, BR, 32), lambda i: (0, i, 0)),
            pl.BlockSpec((64, 32), lambda i: (0, 0)),
            pl.BlockSpec((1, 32), lambda i: (0, 0)),
            pl.BlockSpec((32, 2), lambda i: (0, 0)),
            pl.BlockSpec((1, 2), lambda i: (0, 0)),
        ],
        out_specs=pl.BlockSpec((BR, 2), lambda i: (i, 0)),
        out_shape=jax.ShapeDtypeStruct((N, 2), F32),
        interpret=interpret,
    )


_tc1 = _build_tc1()
_tc2 = _build_tc2(64)
_tc3 = _build_tc3()
_tc4 = _build_tc4()
_tc5 = _build_tc5()
_tc6 = _build_tc2(64)
_tc7 = _build_tc7()


def kernel(x, edge_index, edge_weight, W1r, b1r, W1n, g1, beta1, W2r, b2r,
           W2n, g2, beta2, W3r, b3r, W3n, g3, beta3, Wc1, bc1, Wc2, bc2):
    pad = E_PAD - E
    src = jnp.concatenate([edge_index[0], jnp.zeros((pad,), jnp.int32)])
    dst = jnp.concatenate(
        [edge_index[1], N + (jnp.arange(pad, dtype=jnp.int32) % 16)])
    ew = jnp.concatenate([edge_weight, jnp.zeros((pad,), F32)])

    sc_plain = _build_sc_agg(False)
    sc_weighted = _build_sc_agg(True)
    y1, xn1 = _tc1(x, W1r, W1n)
    agg1 = sc_plain(y1.reshape(2 * N, 32), src, dst, ew)
    z1, st1 = _tc2(agg1, xn1, b1r[None])
    h1, z2p = _tc3(z1, st1, g1[None], beta1[None], W2n)
    agg2 = sc_plain(h1.reshape(2 * N, 32), src, dst, ew)
    z2, st2 = _tc4(agg2, z2p, W2r, b2r[None])
    y3, xn3 = _tc5(z2, st2, g2[None], beta2[None], W3r, W3n)
    agg3 = sc_weighted(y3.reshape(2 * N, 32), src, dst, ew)
    z3, st3 = _tc6(agg3, xn3, b3r[None])
    return _tc7(z3, st3, g3[None], beta3[None], h1, Wc1, bc1[None], Wc2,
                bc2[None])


# speedup vs baseline: 6.0738x; 1.0054x over previous
import functools

import jax
import jax.numpy as jnp
from jax import lax
from jax.experimental import pallas as pl
from jax.experimental.pallas import tpu as pltpu
from jax.experimental.pallas import tpu_sc as plsc

N = 50000
E = 800000
BR = 400
NB = N // BR
CHUNK = 128
N_TILES = 16
N_CHUNKS = 392
EDGES_PER_TILE = N_CHUNKS * CHUNK
E_PAD = N_TILES * EDGES_PER_TILE
ROWS_PER_TILE = 3200
N_ACC = N_TILES * ROWS_PER_TILE
F32 = jnp.float32


def _sc_agg_body(use_ew, preload, stats, table, src, dst, ew, init, out,
                 st_out, src_i, dst_i, ew_b, msg, acc, st_spm, st_buf,
                 ssem, dsem, esem, gsem):
    c = lax.axis_index("c")
    s = lax.axis_index("s")
    r0 = s * ROWS_PER_TILE

    if preload:
        n_full = jnp.maximum(
            0, jnp.minimum(ROWS_PER_TILE // CHUNK, (N - r0) // CHUNK))

        def _pre(k, _):
            r = r0 + k * CHUNK
            pltpu.sync_copy(init.at[c, pl.ds(r, CHUNK)], msg.at[0])
            pltpu.sync_copy(msg.at[0], acc.at[pl.ds(r, CHUNK)])
            return 0
        lax.fori_loop(0, n_full, _pre, 0)
        rem = N - (r0 + n_full * CHUNK)
        has_part = jnp.logical_and(rem > 0, rem < CHUNK)

        @pl.when(has_part)
        def _():
            r = r0 + n_full * CHUNK
            pltpu.sync_copy(init.at[c, pl.ds(r, N % CHUNK)],
                            msg.at[0, pl.ds(0, N % CHUNK)])
            pltpu.sync_copy(msg.at[0, pl.ds(0, N % CHUNK)],
                            acc.at[pl.ds(r, N % CHUNK)])
    else:
        def _zrow(i, _):
            msg[0, i, pl.ds(0, 16)] = jnp.zeros((16,), F32)
            msg[0, i, pl.ds(16, 16)] = jnp.zeros((16,), F32)
            return 0
        lax.fori_loop(0, CHUNK, _zrow, 0)

        def _zacc(k, _):
            pltpu.sync_copy(msg.at[0], acc.at[pl.ds(r0 + k * CHUNK, CHUNK)])
            return 0
        lax.fori_loop(0, ROWS_PER_TILE // CHUNK, _zacc, 0)
    plsc.subcore_barrier()

    base = s * EDGES_PER_TILE
    coff = c * N

    def start_idx(j, b):
        off = base + j * CHUNK
        pltpu.async_copy(src.at[pl.ds(off, CHUNK)], src_i.at[b], ssem.at[b])
        pltpu.async_copy(dst.at[pl.ds(off, CHUNK)], dst_i.at[b], dsem.at[b])
        if use_ew:
            pltpu.async_copy(ew.at[pl.ds(off, CHUNK)], ew_b.at[b],
                             esem.at[b])

    def gather_chunk(b):
        pltpu.make_async_copy(src.at[pl.ds(base, CHUNK)], src_i.at[b],
                              ssem.at[b]).wait()
        for f in range(CHUNK // 16):
            sl = pl.ds(f * 16, 16)
            src_i[b, sl] = src_i[b, sl] + coff
        pltpu.async_copy(table.at[src_i.at[b]], msg.at[b], gsem.at[b])

    def wait_gather(b):
        pltpu.make_async_copy(table.at[src_i.at[b]], msg.at[b],
                              gsem.at[b]).wait()

    def do_scatter(b):
        if use_ew:
            pltpu.make_async_copy(ew.at[pl.ds(base, CHUNK)], ew_b.at[b],
                                  esem.at[b]).wait()
            for g in range(CHUNK // 16):
                w16 = ew_b[b, pl.ds(g * 16, 16)]
                for el in range(16):
                    e = g * 16 + el
                    wv = jnp.full((16,), w16[el], F32)
                    msg[b, e, pl.ds(0, 16)] = msg[b, e, pl.ds(0, 16)] * wv
                    msg[b, e, pl.ds(16, 16)] = msg[b, e, pl.ds(16, 16)] * wv
        pltpu.make_async_copy(dst.at[pl.ds(base, CHUNK)], dst_i.at[b],
                              dsem.at[b]).wait()
        pltpu.sync_copy(msg.at[b], acc.at[dst_i.at[b]], add=True)

    start_idx(0, 0)
    start_idx(1, 1)
    gather_chunk(0)

    def _outer(t, _):
        j0 = 2 * t
        for b in range(2):
            wait_gather(b)
            gather_chunk(1 - b)
            do_scatter(b)
            start_idx(j0 + b + 2, b)
        return 0
    lax.fori_loop(0, (N_CHUNKS - 2) // 2, _outer, 0)
    wait_gather(0)
    gather_chunk(1)
    do_scatter(0)
    wait_gather(1)
    do_scatter(1)
    plsc.subcore_barrier()

    if stats:
        def _drain(k, carry):
            s0, s1, q0, q1 = carry
            r = r0 + k * CHUNK
            pltpu.sync_copy(acc.at[pl.ds(r, CHUNK)], msg.at[0])
            rl = jnp.maximum(0, jnp.minimum(CHUNK, N - r))

            def _row(i, cr):
                a0, a1, b0, b1 = cr
                v0 = msg[0, i, pl.ds(0, 16)]
                v1 = msg[0, i, pl.ds(16, 16)]
                return (a0 + v0, a1 + v1, b0 + v0 * v0, b1 + v1 * v1)
            s0, s1, q0, q1 = lax.fori_loop(0, rl, _row, (s0, s1, q0, q1))
            pltpu.sync_copy(msg.at[0], out.at[c, pl.ds(r, CHUNK)])
            return (s0, s1, q0, q1)
        z16 = jnp.zeros((16,), F32)
        s0, s1, q0, q1 = lax.fori_loop(0, ROWS_PER_TILE // CHUNK, _drain,
                                       (z16, z16, z16, z16))
        st_buf[0, 0] = s0
        st_buf[0, 1] = s1
        st_buf[1, 0] = q0
        st_buf[1, 1] = q1
        pltpu.sync_copy(st_buf, st_spm.at[s])
        plsc.subcore_barrier()

        @pl.when(s == 0)
        def _():
            def _red(t, cr):
                pltpu.sync_copy(st_spm.at[t], st_buf)
                a0, a1, b0, b1 = cr
                return (a0 + st_buf[0, 0], a1 + st_buf[0, 1],
                        b0 + st_buf[1, 0], b1 + st_buf[1, 1])
            rs0, rs1, rq0, rq1 = lax.fori_loop(0, N_TILES, _red,
                                               (z16, z16, z16, z16))
            st_buf[0, 0] = rs0
            st_buf[0, 1] = rs1
            st_buf[1, 0] = rq0
            st_buf[1, 1] = rq1
            pltpu.sync_copy(st_buf.at[0], st_out.at[0, c])
            pltpu.sync_copy(st_buf.at[1], st_out.at[1, c])
    else:
        def _drain(k, _):
            r = r0 + k * CHUNK
            pltpu.sync_copy(acc.at[pl.ds(r, CHUNK)], msg.at[0])
            pltpu.sync_copy(msg.at[0], out.at[c, pl.ds(r, CHUNK)])
            return 0
        lax.fori_loop(0, ROWS_PER_TILE // CHUNK, _drain, 0)


@functools.cache
def _build_sc_agg(use_ew, preload, stats):
    mesh = plsc.VectorSubcoreMesh(core_axis_name="c", subcore_axis_name="s",
                                  num_cores=2, num_subcores=N_TILES)
    return pl.kernel(
        functools.partial(_sc_agg_body, use_ew, preload, stats),
        out_type=[
            jax.ShapeDtypeStruct((2, N_ACC, 32), F32),
            jax.ShapeDtypeStruct((2, 2, 2, 16), F32),
        ],
        mesh=mesh,
        scratch_types=[
            pltpu.VMEM((2, CHUNK), jnp.int32),
            pltpu.VMEM((2, CHUNK), jnp.int32),
            pltpu.VMEM((2, CHUNK), F32),
            pltpu.VMEM((2, CHUNK, 32), F32),
            pltpu.VMEM_SHARED((N_ACC, 32), F32),
            pltpu.VMEM_SHARED((N_TILES, 2, 2, 16), F32),
            pltpu.VMEM((2, 2, 16), F32),
            pltpu.SemaphoreType.DMA((2,)),
            pltpu.SemaphoreType.DMA((2,)),
            pltpu.SemaphoreType.DMA((2,)),
            pltpu.SemaphoreType.DMA((2,)),
        ],
        compiler_params=pltpu.CompilerParams(use_tc_tiling_on_sc=False),
    )


def _tc1_body(x_ref, wr_ref, wn_ref, b_ref, y_ref, xn_ref):
    xb = x_ref[...]
    y = jnp.dot(xb, wr_ref[...], preferred_element_type=F32)
    y_ref[0] = y[:, :32]
    y_ref[1] = y[:, 32:]
    xn = jnp.dot(xb, wn_ref[...], preferred_element_type=F32) + b_ref[...]
    xn_ref[0] = xn[:, :32]
    xn_ref[1] = xn[:, 32:]


def _build_tc1(interpret=False):
    return pl.pallas_call(
        _tc1_body,
        grid=(NB,),
        in_specs=[
            pl.BlockSpec((BR, 128), lambda i: (i, 0)),
            pl.BlockSpec((128, 64), lambda i: (0, 0)),
            pl.BlockSpec((128, 64), lambda i: (0, 0)),
            pl.BlockSpec((1, 64), lambda i: (0, 0)),
        ],
        out_specs=[
            pl.BlockSpec((2, BR, 32), lambda i: (0, i, 0)),
            pl.BlockSpec((2, BR, 32), lambda i: (0, i, 0)),
        ],
        out_shape=[
            jax.ShapeDtypeStruct((2, N, 32), F32),
            jax.ShapeDtypeStruct((2, N, 32), F32),
        ],
        interpret=interpret,
    )


def _sum_stats_body(z, st_ref, i):
    @pl.when(i == 0)
    def _():
        st_ref[...] = jnp.zeros_like(st_ref)
    st_ref[...] += jnp.stack([jnp.sum(z, 0), jnp.sum(z * z, 0)])


def _tc2_body(agg_ref, xn_ref, b_ref, z_ref, st_ref):
    z = jnp.concatenate([agg_ref[0], agg_ref[1]], axis=1) + xn_ref[...] \
        + b_ref[...]
    z_ref[...] = z
    _sum_stats_body(z, st_ref, pl.program_id(0))


def _build_tc2(d, interpret=False):
    return pl.pallas_call(
        _tc2_body,
        grid=(NB,),
        in_specs=[
            pl.BlockSpec((2, BR, 32), lambda i: (0, i, 0)),
            pl.BlockSpec((BR, d), lambda i: (i, 0)),
            pl.BlockSpec((1, d), lambda i: (0, 0)),
        ],
        out_specs=[
            pl.BlockSpec((BR, d), lambda i: (i, 0)),
            pl.BlockSpec((2, d), lambda i: (0, 0)),
        ],
        out_shape=[
            jax.ShapeDtypeStruct((N, d), F32),
            jax.ShapeDtypeStruct((2, d), F32),
        ],
        interpret=interpret,
    )


def _bn_relu(z, st, g, b):
    m = st[0] / N
    v = st[1] / N - m * m
    r = lax.rsqrt(v + 1e-5)
    return jnp.maximum((z - m) * r * g + b, 0.0)


def _tc3_body(z_ref, st_ref, g_ref, b_ref, w_ref, h_ref, zp_ref):
    z = jnp.concatenate([z_ref[0], z_ref[1]], axis=1)
    h = _bn_relu(z, st_ref[...], g_ref[0], b_ref[0])
    h_ref[0] = h[:, :32]
    h_ref[1] = h[:, 32:]
    zp_ref[...] = jnp.dot(h, w_ref[...], preferred_element_type=F32)


def _build_tc3(interpret=False):
    return pl.pallas_call(
        _tc3_body,
        grid=(NB,),
        in_specs=[
            pl.BlockSpec((2, BR, 32), lambda i: (0, i, 0)),
            pl.BlockSpec((2, 64), lambda i: (0, 0)),
            pl.BlockSpec((1, 64), lambda i: (0, 0)),
            pl.BlockSpec((1, 64), lambda i: (0, 0)),
            pl.BlockSpec((64, 128), lambda i: (0, 0)),
        ],
        out_specs=[
            pl.BlockSpec((2, BR, 32), lambda i: (0, i, 0)),
            pl.BlockSpec((BR, 128), lambda i: (i, 0)),
        ],
        out_shape=[
            jax.ShapeDtypeStruct((2, N, 32), F32),
            jax.ShapeDtypeStruct((N, 128), F32),
        ],
        interpret=interpret,
    )


def _tc4_body(agg_ref, zp_ref, wr_ref, b_ref, z_ref, st_ref):
    a = jnp.concatenate([agg_ref[0], agg_ref[1]], axis=1)
    z = jnp.dot(a, wr_ref[...], preferred_element_type=F32) + zp_ref[...] \
        + b_ref[...]
    z_ref[...] = z
    _sum_stats_body(z, st_ref, pl.program_id(0))


def _build_tc4(interpret=False):
    return pl.pallas_call(
        _tc4_body,
        grid=(NB,),
        in_specs=[
            pl.BlockSpec((2, BR, 32), lambda i: (0, i, 0)),
            pl.BlockSpec((BR, 128), lambda i: (i, 0)),
            pl.BlockSpec((64, 128), lambda i: (0, 0)),
            pl.BlockSpec((1, 128), lambda i: (0, 0)),
        ],
        out_specs=[
            pl.BlockSpec((BR, 128), lambda i: (i, 0)),
            pl.BlockSpec((2, 128), lambda i: (0, 0)),
        ],
        out_shape=[
            jax.ShapeDtypeStruct((N, 128), F32),
            jax.ShapeDtypeStruct((2, 128), F32),
        ],
        interpret=interpret,
    )


def _tc5_body(z_ref, st_ref, g_ref, b_ref, wr_ref, wn_ref, bn_ref, y_ref,
              xn_ref):
    h = _bn_relu(z_ref[...], st_ref[...], g_ref[0], b_ref[0])
    y = jnp.dot(h, wr_ref[...], preferred_element_type=F32)
    y_ref[0] = y[:, :32]
    y_ref[1] = y[:, 32:]
    xn = jnp.dot(h, wn_ref[...], preferred_element_type=F32) + bn_ref[...]
    xn_ref[0] = xn[:, :32]
    xn_ref[1] = xn[:, 32:]


def _build_tc5(interpret=False):
    return pl.pallas_call(
        _tc5_body,
        grid=(NB,),
        in_specs=[
            pl.BlockSpec((BR, 128), lambda i: (i, 0)),
            pl.BlockSpec((2, 128), lambda i: (0, 0)),
            pl.BlockSpec((1, 128), lambda i: (0, 0)),
            pl.BlockSpec((1, 128), lambda i: (0, 0)),
            pl.BlockSpec((128, 64), lambda i: (0, 0)),
            pl.BlockSpec((128, 64), lambda i: (0, 0)),
            pl.BlockSpec((1, 64), lambda i: (0, 0)),
        ],
        out_specs=[
            pl.BlockSpec((2, BR, 32), lambda i: (0, i, 0)),
            pl.BlockSpec((2, BR, 32), lambda i: (0, i, 0)),
        ],
        out_shape=[
            jax.ShapeDtypeStruct((2, N, 32), F32),
            jax.ShapeDtypeStruct((2, N, 32), F32),
        ],
        interpret=interpret,
    )


def _tc7_body(z_ref, st_ref, g_ref, b_ref, h1_ref, wc1_ref, bc1_ref, wc2_ref,
              bc2_ref, out_ref):
    st = st_ref[...]
    m = st[0] / N
    v = st[1] / N - m * m
    r = lax.rsqrt(v + 1e-5)
    z = jnp.concatenate([z_ref[0], z_ref[1]], axis=1)
    bn = (z - m) * r * g_ref[0] + b_ref[0]
    h1 = jnp.concatenate([h1_ref[0], h1_ref[1]], axis=1)
    h3 = jnp.maximum(bn + h1, 0.0)
    t = jnp.maximum(jnp.dot(h3, wc1_ref[...], preferred_element_type=F32)
                    + bc1_ref[0], 0.0)
    out_ref[...] = jnp.dot(t, wc2_ref[...], preferred_element_type=F32) \
        + bc2_ref[0]


def _build_tc7(interpret=False):
    return pl.pallas_call(
        _tc7_body,
        grid=(NB,),
        in_specs=[
            pl.BlockSpec((2, BR, 32), lambda i: (0, i, 0)),
            pl.BlockSpec((2, 64), lambda i: (0, 0)),
            pl.BlockSpec((1, 64), lambda i: (0, 0)),
            pl.BlockSpec((1, 64), lambda i: (0, 0)),
            pl.BlockSpec((2, BR, 32), lambda i: (0, i, 0)),
            pl.BlockSpec((64, 32), lambda i: (0, 0)),
            pl.BlockSpec((1, 32), lambda i: (0, 0)),
            pl.BlockSpec((32, 2), lambda i: (0, 0)),
            pl.BlockSpec((1, 2), lambda i: (0, 0)),
        ],
        out_specs=pl.BlockSpec((BR, 2), lambda i: (i, 0)),
        out_shape=jax.ShapeDtypeStruct((N, 2), F32),
        interpret=interpret,
    )


_tc1 = _build_tc1()
_tc3 = _build_tc3()
_tc4 = _build_tc4()
_tc5 = _build_tc5()
_tc7 = _build_tc7()


def kernel(x, edge_index, edge_weight, W1r, b1r, W1n, g1, beta1, W2r, b2r,
           W2n, g2, beta2, W3r, b3r, W3n, g3, beta3, Wc1, bc1, Wc2, bc2):
    pad = E_PAD - E
    src = jnp.concatenate([edge_index[0], jnp.zeros((pad,), jnp.int32)])
    dst = jnp.concatenate(
        [edge_index[1], N + (jnp.arange(pad, dtype=jnp.int32) % 16)])
    ew = jnp.concatenate([edge_weight, jnp.zeros((pad,), F32)])

    sc_z = _build_sc_agg(False, True, True)
    sc_agg = _build_sc_agg(False, False, False)
    sc_z_ew = _build_sc_agg(True, True, True)
    y1, xn1 = _tc1(x, W1r, W1n, b1r[None])
    z1, st1 = sc_z(y1.reshape(2 * N, 32), src, dst, ew, xn1)
    h1, z2p = _tc3(z1, st1.reshape(2, 64), g1[None], beta1[None], W2n)
    agg2, _ = sc_agg(h1.reshape(2 * N, 32), src, dst, ew, xn1)
    z2, st2 = _tc4(agg2, z2p, W2r, b2r[None])
    y3, xn3 = _tc5(z2, st2, g2[None], beta2[None], W3r, W3n, b3r[None])
    z3, st3 = sc_z_ew(y3.reshape(2 * N, 32), src, dst, ew, xn3)
    return _tc7(z3, st3.reshape(2, 64), g3[None], beta3[None], h1, Wc1,
                bc1[None], Wc2, bc2[None])
